# baseline (speedup 1.0000x reference)
import functools

import jax
import jax.numpy as jnp
from jax import lax
from jax.experimental import pallas as pl
from jax.experimental.pallas import tpu as pltpu
from jax.experimental.pallas import tpu_sc as plsc

_N = 10000
_E = 320000
_D = 128
_H = 32
_L = 8
_G = 64

_NC = 2
_NS = 16
_CHUNK = 100
_CPW = 100
_N_PAD = 10240
_RPS = _N_PAD // _NS


def _sc_mesh():
    return plsc.VectorSubcoreMesh(core_axis_name="c", subcore_axis_name="s")


def _sc_deg(edge3d, ones_n, zeros_n):
    @functools.partial(
        pl.kernel,
        out_type=jax.ShapeDtypeStruct((_NC, _N_PAD), jnp.float32),
        mesh=_sc_mesh(),
        compiler_params=pltpu.CompilerParams(use_tc_tiling_on_sc=False),
        scratch_types=[
            pltpu.VMEM((_CPW, _CHUNK), jnp.int32),
            pltpu.VMEM((_CHUNK,), jnp.float32),
            pltpu.VMEM_SHARED((_N_PAD,), jnp.float32),
        ],
    )
    def k(edge_hbm, ones_hbm, zeros_hbm, out_hbm, dst_v, ones_v, deg_sh):
        c = lax.axis_index("c")
        s = lax.axis_index("s")
        wid = c * _NS + s
        pltpu.sync_copy(edge_hbm.at[1, pl.ds(wid * _CPW, _CPW)], dst_v)
        pltpu.sync_copy(ones_hbm, ones_v)
        pltpu.sync_copy(zeros_hbm.at[pl.ds(s * _RPS, _RPS)],
                        deg_sh.at[pl.ds(s * _RPS, _RPS)])
        plsc.subcore_barrier()

        def body(j, carry):
            pltpu.sync_copy(ones_v, deg_sh.at[dst_v.at[j]], add=True)
            return carry

        lax.fori_loop(0, _CPW, body, 0)
        plsc.subcore_barrier()
        pltpu.sync_copy(deg_sh.at[pl.ds(s * _RPS, _RPS)],
                        out_hbm.at[c, pl.ds(s * _RPS, _RPS)])

    return k(edge3d, ones_n, zeros_n)


def _sc_scatter(edge3d, y, zeros_rows):
    @functools.partial(
        pl.kernel,
        out_type=jax.ShapeDtypeStruct((_NC, _N_PAD, _H), jnp.float32),
        mesh=_sc_mesh(),
        compiler_params=pltpu.CompilerParams(use_tc_tiling_on_sc=False),
        scratch_types=[
            pltpu.VMEM((_CPW, _CHUNK), jnp.int32),
            pltpu.VMEM((_CPW, _CHUNK), jnp.int32),
            [pltpu.VMEM((_CHUNK, _H), jnp.float32)] * 4,
            pltpu.VMEM_SHARED((_N_PAD, _H), jnp.float32),
            pltpu.VMEM_SHARED((_N_PAD, _H), jnp.float32),
            [pltpu.SemaphoreType.DMA] * 4,
            [pltpu.SemaphoreType.DMA] * 4,
        ],
    )
    def k(edge_hbm, y_hbm, zeros_hbm, out_hbm,
          src_v, dst_v, rows, acc_sh, y_sh, gsem, ssem):
        c = lax.axis_index("c")
        s = lax.axis_index("s")
        wid = c * _NS + s
        pltpu.sync_copy(edge_hbm.at[0, pl.ds(wid * _CPW, _CPW)], src_v)
        pltpu.sync_copy(edge_hbm.at[1, pl.ds(wid * _CPW, _CPW)], dst_v)
        pltpu.sync_copy(zeros_hbm.at[pl.ds(s * _RPS, _RPS)],
                        acc_sh.at[pl.ds(s * _RPS, _RPS)])
        pltpu.sync_copy(y_hbm.at[pl.ds(s * _RPS, _RPS)],
                        y_sh.at[pl.ds(s * _RPS, _RPS)])
        plsc.subcore_barrier()

        def drain(semref):
            pltpu.make_async_copy(y_hbm.at[pl.ds(0, _CHUNK)],
                                  rows[0], semref).wait()

        pltpu.async_copy(y_sh.at[src_v.at[0]], rows[0], gsem[0])
        pltpu.async_copy(y_sh.at[src_v.at[1]], rows[1], gsem[1])

        def group(gi, carry):
            for b in range(4):
                j = 4 * gi + b
                bb = (b + 2) % 4
                drain(gsem[b])
                pltpu.async_copy(rows[b], acc_sh.at[dst_v.at[j]],
                                 ssem[b], add=True)

                @pl.when(j >= 2)
                def _():
                    drain(ssem[bb])

                @pl.when(j + 2 < _CPW)
                def _():
                    pltpu.async_copy(y_sh.at[src_v.at[j + 2]],
                                     rows[bb], gsem[bb])
            return carry

        lax.fori_loop(0, _CPW // 4, group, 0)
        drain(ssem[(_CPW - 2) % 4])
        drain(ssem[(_CPW - 1) % 4])
        plsc.subcore_barrier()
        pltpu.sync_copy(acc_sh.at[pl.ds(s * _RPS, _RPS)],
                        out_hbm.at[c, pl.ds(s * _RPS, _RPS)])

    return k(edge3d, y, zeros_rows)


def _tc_xw(x_pad, W_gc, deg_parts):
    R = _N_PAD // 4

    def body(x_ref, w_ref, deg_ref, y_ref, dinv_ref):
        xw = jnp.dot(x_ref[...], w_ref[...], preferred_element_type=jnp.float32)
        deg = deg_ref[0, :] + deg_ref[1, :] + 1.0
        dinv = lax.rsqrt(deg)
        y_ref[...] = xw * dinv[:, None]
        dinv_ref[...] = dinv[:, None]

    return pl.pallas_call(
        body,
        grid=(4,),
        in_specs=[
            pl.BlockSpec((R, _D), lambda i: (i, 0)),
            pl.BlockSpec((_D, _H), lambda i: (0, 0)),
            pl.BlockSpec((_NC, R), lambda i: (0, i)),
        ],
        out_specs=[
            pl.BlockSpec((R, _H), lambda i: (i, 0)),
            pl.BlockSpec((R, 1), lambda i: (i, 0)),
        ],
        out_shape=[
            jax.ShapeDtypeStruct((_N_PAD, _H), jnp.float32),
            jax.ShapeDtypeStruct((_N_PAD, 1), jnp.float32),
        ],
    )(x_pad, W_gc, deg_parts)


def _tc_finish(acc_parts, y, dinv, b_gc, batch2d,
               W_mu, b_mu, W_lv, b_lv, W_dec, b_dec, eps):
    R = _N_PAD // 4
    nb = 4

    def body(acc_ref, y_ref, dinv_ref, bgc_ref, batch_ref,
             wmu_ref, bmu_ref, wlv_ref, blv_ref, wdec_ref, bdec_ref, eps_ref,
             adj_ref, mu_ref, lv_ref, seg_ref, cnt_ref):
        i = pl.program_id(0)
        acc = acc_ref[0] + acc_ref[1] + y_ref[...]
        h = jnp.maximum(acc * dinv_ref[...] + bgc_ref[...], 0.0)
        onehot = (batch_ref[...] ==
                  lax.broadcasted_iota(jnp.int32, (_G, R), 0)).astype(jnp.float32)
        sums = lax.dot_general(onehot, h, (((1,), (0,)), ((), ())),
                               preferred_element_type=jnp.float32)
        cnt = jnp.sum(onehot, axis=1, keepdims=True)

        @pl.when(i == 0)
        def _():
            seg_ref[...] = sums
            cnt_ref[...] = cnt

        @pl.when(i > 0)
        def _():
            seg_ref[...] += sums
            cnt_ref[...] += cnt

        @pl.when(i == nb - 1)
        def _():
            hm = seg_ref[...] / jnp.maximum(cnt_ref[...], 1.0)
            mu = jnp.dot(hm, wmu_ref[...],
                         preferred_element_type=jnp.float32) + bmu_ref[...]
            lv = jnp.dot(hm, wlv_ref[...],
                         preferred_element_type=jnp.float32) + blv_ref[...]
            z = mu + eps_ref[...] * jnp.exp(0.5 * lv)
            logits = jnp.dot(z, wdec_ref[...],
                             preferred_element_type=jnp.float32) + bdec_ref[...]
            adj_ref[...] = (logits > 0.0).astype(jnp.float32)
            mu_ref[...] = mu
            lv_ref[...] = lv

    return pl.pallas_call(
        body,
        grid=(nb,),
        in_specs=[
            pl.BlockSpec((_NC, R, _H), lambda i: (0, i, 0)),
            pl.BlockSpec((R, _H), lambda i: (i, 0)),
            pl.BlockSpec((R, 1), lambda i: (i, 0)),
            pl.BlockSpec((1, _H), lambda i: (0, 0)),
            pl.BlockSpec((1, R), lambda i: (0, i)),
            pl.BlockSpec((_H, _L), lambda i: (0, 0)),
            pl.BlockSpec((1, _L), lambda i: (0, 0)),
            pl.BlockSpec((_H, _L), lambda i: (0, 0)),
            pl.BlockSpec((1, _L), lambda i: (0, 0)),
            pl.BlockSpec((_L, 100), lambda i: (0, 0)),
            pl.BlockSpec((1, 100), lambda i: (0, 0)),
            pl.BlockSpec((_G, _L), lambda i: (0, 0)),
        ],
        out_specs=[
            pl.BlockSpec((_G, 100), lambda i: (0, 0)),
            pl.BlockSpec((_G, _L), lambda i: (0, 0)),
            pl.BlockSpec((_G, _L), lambda i: (0, 0)),
        ],
        out_shape=[
            jax.ShapeDtypeStruct((_G, 100), jnp.float32),
            jax.ShapeDtypeStruct((_G, _L), jnp.float32),
            jax.ShapeDtypeStruct((_G, _L), jnp.float32),
        ],
        scratch_shapes=[
            pltpu.VMEM((_G, _H), jnp.float32),
            pltpu.VMEM((_G, 1), jnp.float32),
        ],
    )(acc_parts, y, dinv, b_gc, batch2d,
      W_mu, b_mu, W_lv, b_lv, W_dec, b_dec, eps)


def kernel(x, edge_index, batch, W_gc, b_gc, W_mu, b_mu, W_lv, b_lv,
           W_dec, b_dec, eps):
    edge3d = edge_index.reshape(2, _NC * _NS * _CPW, _CHUNK)
    x_pad = jnp.pad(x, ((0, _N_PAD - _N), (0, 0)))
    batch2d = jnp.pad(batch, (0, _N_PAD - _N),
                      constant_values=_G).reshape(1, _N_PAD)
    ones_n = jnp.ones((_CHUNK,), jnp.float32)
    zeros_n = jnp.zeros((_N_PAD,), jnp.float32)
    zeros_rows = jnp.zeros((_N_PAD, _H), jnp.float32)

    deg_parts = _sc_deg(edge3d, ones_n, zeros_n)
    y, dinv = _tc_xw(x_pad, W_gc, deg_parts)
    acc_parts = _sc_scatter(edge3d, y, zeros_rows)
    adj, mu, lv = _tc_finish(
        acc_parts, y, dinv, b_gc.reshape(1, _H), batch2d,
        W_mu, b_mu.reshape(1, _L), W_lv, b_lv.reshape(1, _L),
        W_dec, b_dec.reshape(1, 100), eps)
    return (adj.reshape(_G, 10, 10), mu, lv)


# speedup vs baseline: 1.0149x; 1.0149x over previous
import functools

import jax
import jax.numpy as jnp
from jax import lax
from jax.experimental import pallas as pl
from jax.experimental.pallas import tpu as pltpu
from jax.experimental.pallas import tpu_sc as plsc

_N = 10000
_E = 320000
_D = 128
_H = 32
_L = 8
_G = 64

_NC = 2
_NS = 16
_CHUNK = 125
_CPW = 80
_N_PAD = 10240
_RPS = _N_PAD // _NS


def _sc_mesh():
    return plsc.VectorSubcoreMesh(core_axis_name="c", subcore_axis_name="s")


def _sc_deg(dst2d, ones_n, zeros_n):
    @functools.partial(
        pl.kernel,
        out_type=jax.ShapeDtypeStruct((_NC, _N_PAD), jnp.float32),
        mesh=_sc_mesh(),
        compiler_params=pltpu.CompilerParams(use_tc_tiling_on_sc=False),
        scratch_types=[
            pltpu.VMEM((_CPW, _CHUNK), jnp.int32),
            pltpu.VMEM((_CHUNK,), jnp.float32),
            pltpu.VMEM_SHARED((_N_PAD,), jnp.float32),
        ],
    )
    def k(dst_hbm, ones_hbm, zeros_hbm, out_hbm, dst_v, ones_v, deg_sh):
        c = lax.axis_index("c")
        s = lax.axis_index("s")
        wid = c * _NS + s
        pltpu.sync_copy(dst_hbm.at[pl.ds(wid * _CPW, _CPW)], dst_v)
        pltpu.sync_copy(ones_hbm, ones_v)
        pltpu.sync_copy(zeros_hbm.at[pl.ds(s * _RPS, _RPS)],
                        deg_sh.at[pl.ds(s * _RPS, _RPS)])
        plsc.subcore_barrier()

        def body(j, carry):
            pltpu.sync_copy(ones_v, deg_sh.at[dst_v.at[j]], add=True)
            return carry

        lax.fori_loop(0, _CPW, body, 0)
        plsc.subcore_barrier()
        pltpu.sync_copy(deg_sh.at[pl.ds(s * _RPS, _RPS)],
                        out_hbm.at[c, pl.ds(s * _RPS, _RPS)])

    return k(dst2d, ones_n, zeros_n)


def _sc_scatter(src2d, dst2d, y, zeros_rows):
    @functools.partial(
        pl.kernel,
        out_type=jax.ShapeDtypeStruct((_NC, _N_PAD, _H), jnp.float32),
        mesh=_sc_mesh(),
        compiler_params=pltpu.CompilerParams(use_tc_tiling_on_sc=False),
        scratch_types=[
            pltpu.VMEM((_CPW, _CHUNK), jnp.int32),
            pltpu.VMEM((_CPW, _CHUNK), jnp.int32),
            [pltpu.VMEM((_CHUNK, _H), jnp.float32)] * 4,
            pltpu.VMEM_SHARED((_N_PAD, _H), jnp.float32),
            pltpu.VMEM_SHARED((_N_PAD, _H), jnp.float32),
            [pltpu.SemaphoreType.DMA] * 4,
            [pltpu.SemaphoreType.DMA] * 4,
        ],
    )
    def k(src_hbm, dst_hbm, y_hbm, zeros_hbm, out_hbm,
          src_v, dst_v, rows, acc_sh, y_sh, gsem, ssem):
        c = lax.axis_index("c")
        s = lax.axis_index("s")
        wid = c * _NS + s
        pltpu.sync_copy(src_hbm.at[pl.ds(wid * _CPW, _CPW)], src_v)
        pltpu.sync_copy(dst_hbm.at[pl.ds(wid * _CPW, _CPW)], dst_v)
        pltpu.sync_copy(zeros_hbm.at[pl.ds(s * _RPS, _RPS)],
                        acc_sh.at[pl.ds(s * _RPS, _RPS)])
        pltpu.sync_copy(y_hbm.at[pl.ds(s * _RPS, _RPS)],
                        y_sh.at[pl.ds(s * _RPS, _RPS)])
        plsc.subcore_barrier()

        def drain(semref):
            pltpu.make_async_copy(y_hbm.at[pl.ds(0, _CHUNK)],
                                  rows[0], semref).wait()

        pltpu.async_copy(y_sh.at[src_v.at[0]], rows[0], gsem[0])
        pltpu.async_copy(y_sh.at[src_v.at[1]], rows[1], gsem[1])

        def group(gi, carry):
            for b in range(4):
                j = 4 * gi + b
                bb = (b + 2) % 4
                drain(gsem[b])
                pltpu.async_copy(rows[b], acc_sh.at[dst_v.at[j]],
                                 ssem[b], add=True)

                @pl.when(j >= 2)
                def _():
                    drain(ssem[bb])

                @pl.when(j + 2 < _CPW)
                def _():
                    pltpu.async_copy(y_sh.at[src_v.at[j + 2]],
                                     rows[bb], gsem[bb])
            return carry

        lax.fori_loop(0, _CPW // 4, group, 0)
        drain(ssem[(_CPW - 2) % 4])
        drain(ssem[(_CPW - 1) % 4])
        plsc.subcore_barrier()
        pltpu.sync_copy(acc_sh.at[pl.ds(s * _RPS, _RPS)],
                        out_hbm.at[c, pl.ds(s * _RPS, _RPS)])

    return k(src2d, dst2d, y, zeros_rows)


def _tc_xw(x_pad, W_gc, deg_parts):
    R = _N_PAD // 4

    def body(x_ref, w_ref, deg_ref, y_ref, dinv_ref):
        xw = jnp.dot(x_ref[...], w_ref[...], preferred_element_type=jnp.float32)
        deg = deg_ref[0, :] + deg_ref[1, :] + 1.0
        dinv = lax.rsqrt(deg)
        y_ref[...] = xw * dinv[:, None]
        dinv_ref[...] = dinv[:, None]

    return pl.pallas_call(
        body,
        grid=(4,),
        in_specs=[
            pl.BlockSpec((R, _D), lambda i: (i, 0)),
            pl.BlockSpec((_D, _H), lambda i: (0, 0)),
            pl.BlockSpec((_NC, R), lambda i: (0, i)),
        ],
        out_specs=[
            pl.BlockSpec((R, _H), lambda i: (i, 0)),
            pl.BlockSpec((R, 1), lambda i: (i, 0)),
        ],
        out_shape=[
            jax.ShapeDtypeStruct((_N_PAD, _H), jnp.float32),
            jax.ShapeDtypeStruct((_N_PAD, 1), jnp.float32),
        ],
    )(x_pad, W_gc, deg_parts)


def _tc_finish(acc_parts, y, dinv, b_gc, batch2d,
               W_mu, b_mu, W_lv, b_lv, W_dec, b_dec, eps):
    R = _N_PAD // 4
    nb = 4

    def body(acc_ref, y_ref, dinv_ref, bgc_ref, batch_ref,
             wmu_ref, bmu_ref, wlv_ref, blv_ref, wdec_ref, bdec_ref, eps_ref,
             adj_ref, mu_ref, lv_ref, seg_ref, cnt_ref):
        i = pl.program_id(0)
        acc = acc_ref[0] + acc_ref[1] + y_ref[...]
        h = jnp.maximum(acc * dinv_ref[...] + bgc_ref[...], 0.0)
        onehot = (batch_ref[...] ==
                  lax.broadcasted_iota(jnp.int32, (_G, R), 0)).astype(jnp.float32)
        sums = lax.dot_general(onehot, h, (((1,), (0,)), ((), ())),
                               preferred_element_type=jnp.float32)
        cnt = jnp.sum(onehot, axis=1, keepdims=True)

        @pl.when(i == 0)
        def _():
            seg_ref[...] = sums
            cnt_ref[...] = cnt

        @pl.when(i > 0)
        def _():
            seg_ref[...] += sums
            cnt_ref[...] += cnt

        @pl.when(i == nb - 1)
        def _():
            hm = seg_ref[...] / jnp.maximum(cnt_ref[...], 1.0)
            mu = jnp.dot(hm, wmu_ref[...],
                         preferred_element_type=jnp.float32) + bmu_ref[...]
            lv = jnp.dot(hm, wlv_ref[...],
                         preferred_element_type=jnp.float32) + blv_ref[...]
            z = mu + eps_ref[...] * jnp.exp(0.5 * lv)
            logits = jnp.dot(z, wdec_ref[...],
                             preferred_element_type=jnp.float32) + bdec_ref[...]
            adj_ref[...] = (logits > 0.0).astype(jnp.float32)
            mu_ref[...] = mu
            lv_ref[...] = lv

    return pl.pallas_call(
        body,
        grid=(nb,),
        in_specs=[
            pl.BlockSpec((_NC, R, _H), lambda i: (0, i, 0)),
            pl.BlockSpec((R, _H), lambda i: (i, 0)),
            pl.BlockSpec((R, 1), lambda i: (i, 0)),
            pl.BlockSpec((1, _H), lambda i: (0, 0)),
            pl.BlockSpec((1, R), lambda i: (0, i)),
            pl.BlockSpec((_H, _L), lambda i: (0, 0)),
            pl.BlockSpec((1, _L), lambda i: (0, 0)),
            pl.BlockSpec((_H, _L), lambda i: (0, 0)),
            pl.BlockSpec((1, _L), lambda i: (0, 0)),
            pl.BlockSpec((_L, 100), lambda i: (0, 0)),
            pl.BlockSpec((1, 100), lambda i: (0, 0)),
            pl.BlockSpec((_G, _L), lambda i: (0, 0)),
        ],
        out_specs=[
            pl.BlockSpec((_G, 100), lambda i: (0, 0)),
            pl.BlockSpec((_G, _L), lambda i: (0, 0)),
            pl.BlockSpec((_G, _L), lambda i: (0, 0)),
        ],
        out_shape=[
            jax.ShapeDtypeStruct((_G, 100), jnp.float32),
            jax.ShapeDtypeStruct((_G, _L), jnp.float32),
            jax.ShapeDtypeStruct((_G, _L), jnp.float32),
        ],
        scratch_shapes=[
            pltpu.VMEM((_G, _H), jnp.float32),
            pltpu.VMEM((_G, 1), jnp.float32),
        ],
    )(acc_parts, y, dinv, b_gc, batch2d,
      W_mu, b_mu, W_lv, b_lv, W_dec, b_dec, eps)


def kernel(x, edge_index, batch, W_gc, b_gc, W_mu, b_mu, W_lv, b_lv,
           W_dec, b_dec, eps):
    src2d = edge_index[0].reshape(_NC * _NS * _CPW, _CHUNK)
    dst2d = edge_index[1].reshape(_NC * _NS * _CPW, _CHUNK)
    x_pad = jnp.pad(x, ((0, _N_PAD - _N), (0, 0)))
    batch2d = jnp.pad(batch, (0, _N_PAD - _N),
                      constant_values=_G).reshape(1, _N_PAD)
    ones_n = jnp.ones((_CHUNK,), jnp.float32)
    zeros_n = jnp.zeros((_N_PAD,), jnp.float32)
    zeros_rows = jnp.zeros((_N_PAD, _H), jnp.float32)

    deg_parts = _sc_deg(dst2d, ones_n, zeros_n)
    y, dinv = _tc_xw(x_pad, W_gc, deg_parts)
    acc_parts = _sc_scatter(src2d, dst2d, y, zeros_rows)
    adj, mu, lv = _tc_finish(
        acc_parts, y, dinv, b_gc.reshape(1, _H), batch2d,
        W_mu, b_mu.reshape(1, _L), W_lv, b_lv.reshape(1, _L),
        W_dec, b_dec.reshape(1, 100), eps)
    return (adj.reshape(_G, 10, 10), mu, lv)


# speedup vs baseline: 1.0188x; 1.0039x over previous
import functools

import jax
import jax.numpy as jnp
from jax import lax
from jax.experimental import pallas as pl
from jax.experimental.pallas import tpu as pltpu
from jax.experimental.pallas import tpu_sc as plsc

_N = 10000
_E = 320000
_D = 128
_H = 32
_L = 8
_G = 64

_NC = 2
_NS = 16
_CHUNK = 125
_CPW = 80
_N_PAD = 10240
_RPS = _N_PAD // _NS


def _sc_mesh():
    return plsc.VectorSubcoreMesh(core_axis_name="c", subcore_axis_name="s")


def _sc_deg(dst2d, ones_n, zeros_n):
    @functools.partial(
        pl.kernel,
        out_type=jax.ShapeDtypeStruct((_NC, _N_PAD), jnp.float32),
        mesh=_sc_mesh(),
        compiler_params=pltpu.CompilerParams(use_tc_tiling_on_sc=False),
        scratch_types=[
            pltpu.VMEM((_CPW, _CHUNK), jnp.int32),
            pltpu.VMEM((_CHUNK,), jnp.float32),
            pltpu.VMEM_SHARED((_N_PAD,), jnp.float32),
        ],
    )
    def k(dst_hbm, ones_hbm, zeros_hbm, out_hbm, dst_v, ones_v, deg_sh):
        c = lax.axis_index("c")
        s = lax.axis_index("s")
        wid = c * _NS + s
        pltpu.sync_copy(dst_hbm.at[pl.ds(wid * _CPW, _CPW)], dst_v)
        pltpu.sync_copy(ones_hbm, ones_v)
        pltpu.sync_copy(zeros_hbm.at[pl.ds(s * _RPS, _RPS)],
                        deg_sh.at[pl.ds(s * _RPS, _RPS)])
        plsc.subcore_barrier()

        def body(j, carry):
            pltpu.sync_copy(ones_v, deg_sh.at[dst_v.at[j]], add=True)
            return carry

        lax.fori_loop(0, _CPW, body, 0)
        plsc.subcore_barrier()
        pltpu.sync_copy(deg_sh.at[pl.ds(s * _RPS, _RPS)],
                        out_hbm.at[c, pl.ds(s * _RPS, _RPS)])

    return k(dst2d, ones_n, zeros_n)


def _sc_scatter(src2d, dst2d, y, zeros_rows):
    @functools.partial(
        pl.kernel,
        out_type=jax.ShapeDtypeStruct((_NC, _N_PAD, _H), jnp.float32),
        mesh=_sc_mesh(),
        compiler_params=pltpu.CompilerParams(use_tc_tiling_on_sc=False),
        scratch_types=[
            pltpu.VMEM((_CPW, _CHUNK), jnp.int32),
            pltpu.VMEM((_CPW, _CHUNK), jnp.int32),
            [pltpu.VMEM((_CHUNK, _H), jnp.float32)] * 8,
            pltpu.VMEM_SHARED((_N_PAD, _H), jnp.float32),
            pltpu.VMEM_SHARED((_N_PAD, _H), jnp.float32),
            [pltpu.SemaphoreType.DMA] * 8,
            [pltpu.SemaphoreType.DMA] * 8,
        ],
    )
    def k(src_hbm, dst_hbm, y_hbm, zeros_hbm, out_hbm,
          src_v, dst_v, rows, acc_sh, y_sh, gsem, ssem):
        c = lax.axis_index("c")
        s = lax.axis_index("s")
        wid = c * _NS + s
        pltpu.sync_copy(src_hbm.at[pl.ds(wid * _CPW, _CPW)], src_v)
        pltpu.sync_copy(dst_hbm.at[pl.ds(wid * _CPW, _CPW)], dst_v)
        pltpu.sync_copy(zeros_hbm.at[pl.ds(s * _RPS, _RPS)],
                        acc_sh.at[pl.ds(s * _RPS, _RPS)])
        pltpu.sync_copy(y_hbm.at[pl.ds(s * _RPS, _RPS)],
                        y_sh.at[pl.ds(s * _RPS, _RPS)])
        plsc.subcore_barrier()

        def drain(semref):
            pltpu.make_async_copy(y_hbm.at[pl.ds(0, _CHUNK)],
                                  rows[0], semref).wait()

        for b0 in range(4):
            pltpu.async_copy(y_sh.at[src_v.at[b0]], rows[b0], gsem[b0])

        def group(gi, carry):
            for b in range(8):
                j = 8 * gi + b
                bb = (b + 4) % 8
                drain(gsem[b])
                pltpu.async_copy(rows[b], acc_sh.at[dst_v.at[j]],
                                 ssem[b], add=True)

                @pl.when(j >= 4)
                def _():
                    drain(ssem[bb])

                @pl.when(j + 4 < _CPW)
                def _():
                    pltpu.async_copy(y_sh.at[src_v.at[j + 4]],
                                     rows[bb], gsem[bb])
            return carry

        lax.fori_loop(0, _CPW // 8, group, 0)
        for bt in range(4):
            drain(ssem[(_CPW - 4 + bt) % 8])
        plsc.subcore_barrier()
        pltpu.sync_copy(acc_sh.at[pl.ds(s * _RPS, _RPS)],
                        out_hbm.at[c, pl.ds(s * _RPS, _RPS)])

    return k(src2d, dst2d, y, zeros_rows)


def _tc_xw(x_pad, W_gc, deg_parts):
    R = _N_PAD // 4

    def body(x_ref, w_ref, deg_ref, y_ref, dinv_ref):
        xw = jnp.dot(x_ref[...], w_ref[...], preferred_element_type=jnp.float32)
        deg = deg_ref[0, :] + deg_ref[1, :] + 1.0
        dinv = lax.rsqrt(deg)
        y_ref[...] = xw * dinv[:, None]
        dinv_ref[...] = dinv[:, None]

    return pl.pallas_call(
        body,
        grid=(4,),
        in_specs=[
            pl.BlockSpec((R, _D), lambda i: (i, 0)),
            pl.BlockSpec((_D, _H), lambda i: (0, 0)),
            pl.BlockSpec((_NC, R), lambda i: (0, i)),
        ],
        out_specs=[
            pl.BlockSpec((R, _H), lambda i: (i, 0)),
            pl.BlockSpec((R, 1), lambda i: (i, 0)),
        ],
        out_shape=[
            jax.ShapeDtypeStruct((_N_PAD, _H), jnp.float32),
            jax.ShapeDtypeStruct((_N_PAD, 1), jnp.float32),
        ],
    )(x_pad, W_gc, deg_parts)


def _tc_finish(acc_parts, y, dinv, b_gc, batch2d,
               W_mu, b_mu, W_lv, b_lv, W_dec, b_dec, eps):
    R = _N_PAD // 4
    nb = 4

    def body(acc_ref, y_ref, dinv_ref, bgc_ref, batch_ref,
             wmu_ref, bmu_ref, wlv_ref, blv_ref, wdec_ref, bdec_ref, eps_ref,
             adj_ref, mu_ref, lv_ref, seg_ref, cnt_ref):
        i = pl.program_id(0)
        acc = acc_ref[0] + acc_ref[1] + y_ref[...]
        h = jnp.maximum(acc * dinv_ref[...] + bgc_ref[...], 0.0)
        onehot = (batch_ref[...] ==
                  lax.broadcasted_iota(jnp.int32, (_G, R), 0)).astype(jnp.float32)
        sums = lax.dot_general(onehot, h, (((1,), (0,)), ((), ())),
                               preferred_element_type=jnp.float32)
        cnt = jnp.sum(onehot, axis=1, keepdims=True)

        @pl.when(i == 0)
        def _():
            seg_ref[...] = sums
            cnt_ref[...] = cnt

        @pl.when(i > 0)
        def _():
            seg_ref[...] += sums
            cnt_ref[...] += cnt

        @pl.when(i == nb - 1)
        def _():
            hm = seg_ref[...] / jnp.maximum(cnt_ref[...], 1.0)
            mu = jnp.dot(hm, wmu_ref[...],
                         preferred_element_type=jnp.float32) + bmu_ref[...]
            lv = jnp.dot(hm, wlv_ref[...],
                         preferred_element_type=jnp.float32) + blv_ref[...]
            z = mu + eps_ref[...] * jnp.exp(0.5 * lv)
            logits = jnp.dot(z, wdec_ref[...],
                             preferred_element_type=jnp.float32) + bdec_ref[...]
            adj_ref[...] = (logits > 0.0).astype(jnp.float32)
            mu_ref[...] = mu
            lv_ref[...] = lv

    return pl.pallas_call(
        body,
        grid=(nb,),
        in_specs=[
            pl.BlockSpec((_NC, R, _H), lambda i: (0, i, 0)),
            pl.BlockSpec((R, _H), lambda i: (i, 0)),
            pl.BlockSpec((R, 1), lambda i: (i, 0)),
            pl.BlockSpec((1, _H), lambda i: (0, 0)),
            pl.BlockSpec((1, R), lambda i: (0, i)),
            pl.BlockSpec((_H, _L), lambda i: (0, 0)),
            pl.BlockSpec((1, _L), lambda i: (0, 0)),
            pl.BlockSpec((_H, _L), lambda i: (0, 0)),
            pl.BlockSpec((1, _L), lambda i: (0, 0)),
            pl.BlockSpec((_L, 100), lambda i: (0, 0)),
            pl.BlockSpec((1, 100), lambda i: (0, 0)),
            pl.BlockSpec((_G, _L), lambda i: (0, 0)),
        ],
        out_specs=[
            pl.BlockSpec((_G, 100), lambda i: (0, 0)),
            pl.BlockSpec((_G, _L), lambda i: (0, 0)),
            pl.BlockSpec((_G, _L), lambda i: (0, 0)),
        ],
        out_shape=[
            jax.ShapeDtypeStruct((_G, 100), jnp.float32),
            jax.ShapeDtypeStruct((_G, _L), jnp.float32),
            jax.ShapeDtypeStruct((_G, _L), jnp.float32),
        ],
        scratch_shapes=[
            pltpu.VMEM((_G, _H), jnp.float32),
            pltpu.VMEM((_G, 1), jnp.float32),
        ],
    )(acc_parts, y, dinv, b_gc, batch2d,
      W_mu, b_mu, W_lv, b_lv, W_dec, b_dec, eps)


def kernel(x, edge_index, batch, W_gc, b_gc, W_mu, b_mu, W_lv, b_lv,
           W_dec, b_dec, eps):
    src2d = edge_index[0].reshape(_NC * _NS * _CPW, _CHUNK)
    dst2d = edge_index[1].reshape(_NC * _NS * _CPW, _CHUNK)
    x_pad = jnp.pad(x, ((0, _N_PAD - _N), (0, 0)))
    batch2d = jnp.pad(batch, (0, _N_PAD - _N),
                      constant_values=_G).reshape(1, _N_PAD)
    ones_n = jnp.ones((_CHUNK,), jnp.float32)
    zeros_n = jnp.zeros((_N_PAD,), jnp.float32)
    zeros_rows = jnp.zeros((_N_PAD, _H), jnp.float32)

    deg_parts = _sc_deg(dst2d, ones_n, zeros_n)
    y, dinv = _tc_xw(x_pad, W_gc, deg_parts)
    acc_parts = _sc_scatter(src2d, dst2d, y, zeros_rows)
    adj, mu, lv = _tc_finish(
        acc_parts, y, dinv, b_gc.reshape(1, _H), batch2d,
        W_mu, b_mu.reshape(1, _L), W_lv, b_lv.reshape(1, _L),
        W_dec, b_dec.reshape(1, 100), eps)
    return (adj.reshape(_G, 10, 10), mu, lv)
